# Initial kernel scaffold; baseline (speedup 1.0000x reference)
import jax
import jax.numpy as jnp
from jax.experimental import pallas as pl


def kernel(chars, char_embed, pos_embed, to_bulk_w, to_bulk_b):
    raise NotImplementedError("write your pallas kernel here")


# speedup vs baseline: 74.0439x; 74.0439x over previous
import functools

import jax
import jax.numpy as jnp
from jax import lax
from jax.experimental import pallas as pl
from jax.experimental.pallas import tpu as pltpu
from jax.experimental.pallas import tpu_sc as plsc

_NUM_CORES = 2
_NUM_SUBCORES = 16
_LANES = 16
_NUM_WORKERS = _NUM_CORES * _NUM_SUBCORES


def _histogram_sc(chars, vocab):
    bsz, seq = chars.shape
    rows_per = bsz // _NUM_WORKERS
    mesh = plsc.VectorSubcoreMesh(core_axis_name="c", subcore_axis_name="s")

    @functools.partial(
        pl.kernel,
        mesh=mesh,
        out_type=jax.ShapeDtypeStruct((bsz, vocab), jnp.float32),
        compiler_params=pltpu.CompilerParams(needs_layout_passes=False),
        scratch_types=[
            pltpu.VMEM((rows_per, seq), jnp.int32),
            pltpu.VMEM((_LANES * vocab,), jnp.float32),
            pltpu.VMEM((rows_per, vocab), jnp.float32),
        ],
    )
    def hist_kernel(chars_hbm, out_hbm, chars_v, subhist_v, counts_v):
        wid = lax.axis_index("s") * _NUM_CORES + lax.axis_index("c")
        base = wid * rows_per
        pltpu.sync_copy(chars_hbm.at[pl.ds(base, rows_per)], chars_v)

        lane_off = lax.iota(jnp.int32, _LANES) * vocab
        ones = jnp.ones((_LANES,), jnp.float32)
        zeros = jnp.zeros((_LANES,), jnp.float32)
        n_chunks = vocab // _LANES

        def zero_body(i, _):
            subhist_v[pl.ds(i * _LANES, _LANES)] = zeros
            return 0
        lax.fori_loop(0, (_LANES * vocab) // _LANES, zero_body, 0)

        for r in range(rows_per):
            def scatter_body(k, _, r=r):
                vals = chars_v[r, pl.ds(k * _LANES, _LANES)]
                plsc.addupdate_scatter(subhist_v, [lane_off + vals], ones)
                return 0
            lax.fori_loop(0, seq // _LANES, scatter_body, 0)

            def reduce_body(cc, _, r=r):
                col = pl.ds(cc * _LANES, _LANES)
                total = subhist_v[col]
                subhist_v[col] = zeros
                for rr in range(1, _LANES):
                    chunk = pl.ds(rr * vocab + cc * _LANES, _LANES)
                    total = total + subhist_v[chunk]
                    subhist_v[chunk] = zeros
                counts_v[r, col] = total
                return 0
            lax.fori_loop(0, n_chunks, reduce_body, 0)

        pltpu.sync_copy(counts_v, out_hbm.at[pl.ds(base, rows_per)])

    return hist_kernel(chars)


def _dense_tc(counts, char_embed, pos_slice, w_t, bias_row, inv_len):
    bsz = counts.shape[0]
    bulk = w_t.shape[1]

    def dense_kernel(counts_ref, ce_ref, pos_ref, wt_ref, b_ref, out_ref):
        pooled = jnp.dot(counts_ref[...], ce_ref[...],
                         preferred_element_type=jnp.float32)
        pos_sum = jnp.sum(pos_ref[...], axis=0, keepdims=True)
        x = (pooled + pos_sum) * inv_len
        out_ref[...] = jnp.dot(x, wt_ref[...],
                               preferred_element_type=jnp.float32) + b_ref[...]

    return pl.pallas_call(
        dense_kernel,
        out_shape=jax.ShapeDtypeStruct((bsz, bulk), jnp.float32),
    )(counts, char_embed, pos_slice, w_t, bias_row)


def kernel(chars, char_embed, pos_embed, to_bulk_w, to_bulk_b):
    bsz, seq = chars.shape
    vocab, _ = char_embed.shape
    counts = _histogram_sc(chars, vocab)
    pos_slice = pos_embed[:seq]
    w_t = to_bulk_w.T
    bias_row = to_bulk_b.reshape(1, -1)
    return _dense_tc(counts, char_embed, pos_slice, w_t, bias_row, 1.0 / seq)
